# Initial kernel scaffold; baseline (speedup 1.0000x reference)
import functools

import jax
import jax.numpy as jnp
from jax import lax
from jax.experimental import pallas as pl
from jax.experimental.pallas import tpu as pltpu
from jax.experimental.pallas import tpu_sc as plsc

NC = 2
NS = 16
NW = NC * NS


def _make_degree_kernel(npad2, nchunk, k):
    mesh = plsc.VectorSubcoreMesh(core_axis_name="c", subcore_axis_name="s")
    ept = npad2 // NS

    @functools.partial(
        pl.kernel,
        out_type=jax.ShapeDtypeStruct((NC, npad2), jnp.float32),
        mesh=mesh,
        scratch_types=[
            pltpu.VMEM((nchunk, k), jnp.int32),
            pltpu.VMEM((k,), jnp.float32),
            pltpu.VMEM_SHARED((npad2,), jnp.float32),
        ],
    )
    def deg_kernel(idx_hbm, ones_hbm, zer_hbm, out_hbm, idx_v, ones_v, acc):
        cid = lax.axis_index("c")
        sid = lax.axis_index("s")
        wid = sid * NC + cid
        pltpu.sync_copy(zer_hbm.at[pl.ds(sid * ept, ept)],
                        acc.at[pl.ds(sid * ept, ept)])
        pltpu.sync_copy(idx_hbm.at[wid], idx_v)
        pltpu.sync_copy(ones_hbm, ones_v)
        plsc.subcore_barrier()

        def body(j, carry):
            pltpu.sync_copy(ones_v, acc.at[idx_v.at[j]], add=True)
            return carry

        lax.fori_loop(0, nchunk, body, 0)
        plsc.subcore_barrier()
        pltpu.sync_copy(acc.at[pl.ds(sid * ept, ept)],
                        out_hbm.at[cid, pl.ds(sid * ept, ept)])

    return deg_kernel


def _make_gather_scatter_kernel(npad, h, nchunk, k):
    mesh = plsc.VectorSubcoreMesh(core_axis_name="c", subcore_axis_name="s")
    rpt = npad // NS
    nbuf = 2

    @functools.partial(
        pl.kernel,
        out_type=jax.ShapeDtypeStruct((NC, npad, h), jnp.float32),
        mesh=mesh,
        scratch_types=[
            pltpu.VMEM((nchunk, k), jnp.int32),
            pltpu.VMEM((nchunk, k), jnp.int32),
            pltpu.VMEM((2, k, h), jnp.float32),
            pltpu.VMEM_SHARED((npad, h), jnp.float32),
            pltpu.SemaphoreType.DMA,
            pltpu.SemaphoreType.DMA,
        ],
    )
    def gs_kernel(hs_hbm, src_hbm, dst_hbm, zer_hbm, out_hbm,
                  src_v, dst_v, rows_v, acc, sem0, sem1):
        cid = lax.axis_index("c")
        sid = lax.axis_index("s")
        wid = sid * NC + cid
        pltpu.sync_copy(zer_hbm.at[pl.ds(sid * rpt, rpt)],
                        acc.at[pl.ds(sid * rpt, rpt)])
        pltpu.sync_copy(src_hbm.at[wid], src_v)
        pltpu.sync_copy(dst_hbm.at[wid], dst_v)
        plsc.subcore_barrier()

        sems = (sem0, sem1)
        for b in range(nbuf):
            pltpu.async_copy(hs_hbm.at[src_v.at[b]], rows_v.at[b], sems[b])

        def body(j, carry):
            for b in range(nbuf):
                cur = j * nbuf + b
                pltpu.make_async_copy(hs_hbm.at[pl.ds(0, k)],
                                      rows_v.at[b], sems[b]).wait()
                pltpu.sync_copy(rows_v.at[b], acc.at[dst_v.at[cur]], add=True)
                nxt = cur + nbuf

                @pl.when(nxt < nchunk)
                def _issue():
                    pltpu.async_copy(hs_hbm.at[src_v.at[nxt]],
                                     rows_v.at[b], sems[b])
            return carry

        lax.fori_loop(0, nchunk // nbuf, body, 0)
        plsc.subcore_barrier()
        pltpu.sync_copy(acc.at[pl.ds(sid * rpt, rpt)],
                        out_hbm.at[cid, pl.ds(sid * rpt, rpt)])

    return gs_kernel


def _in_body(x_ref, w_ref, deg_ref, h0_ref, hs_ref):
    ds = deg_ref[0] + deg_ref[1]
    r_out = lax.rsqrt(jnp.maximum(ds[0], 1.0))
    hm = jnp.dot(x_ref[...], w_ref[...], preferred_element_type=jnp.float32)
    h0 = jnp.maximum(hm, 0.0)
    h0_ref[...] = h0
    hs_ref[...] = h0 * r_out


def _mid_body(p_ref, h0_ref, deg_ref, w_ref, hs_ref):
    ds = deg_ref[0] + deg_ref[1]
    r_out = lax.rsqrt(jnp.maximum(ds[0], 1.0))
    r_in = lax.rsqrt(jnp.maximum(ds[1], 1.0))
    agg = (p_ref[0] + p_ref[1]) * r_in
    t = jnp.dot(agg + h0_ref[...], w_ref[...],
                preferred_element_type=jnp.float32)
    hs_ref[...] = jnp.maximum(t, 0.0) * r_out


def _out_body(p_ref, h0_ref, deg_ref, w2_ref, wo_ref, o_ref):
    ds = deg_ref[0] + deg_ref[1]
    r_in = lax.rsqrt(jnp.maximum(ds[1], 1.0))
    agg = (p_ref[0] + p_ref[1]) * r_in
    t = jnp.dot(agg + h0_ref[...], w2_ref[...],
                preferred_element_type=jnp.float32)
    o_ref[...] = jnp.dot(jnp.maximum(t, 0.0), wo_ref[...],
                         preferred_element_type=jnp.float32)


def kernel(x, edge_index, W_in, W1, W2, W_out):
    N, F = x.shape
    H = W1.shape[0]
    C = W_out.shape[1]
    E = edge_index.shape[1]

    R = 1024
    NPAD = -(-N // R) * R
    G = NPAD // R

    K = 125
    EW = E // NW
    NCH = EW // K
    EW2 = 2 * E // NW
    NCH2 = EW2 // K

    src = edge_index[0]
    dst = edge_index[1]
    src_r = src.reshape(NW, NCH, K)
    dst_r = dst.reshape(NW, NCH, K)
    idx_all = jnp.concatenate([src, dst + NPAD]).reshape(NW, NCH2, K)

    ones_k = jnp.ones((K,), jnp.float32)
    zeros1 = jnp.zeros((2 * NPAD,), jnp.float32)
    zeros2 = jnp.zeros((NPAD, H), jnp.float32)
    xp = jnp.pad(x, ((0, NPAD - N), (0, 0)))

    deg_call = _make_degree_kernel(2 * NPAD, NCH2, K)
    gs_call = _make_gather_scatter_kernel(NPAD, H, NCH, K)

    degP = deg_call(idx_all, ones_k, zeros1)
    deg4 = degP.reshape(NC, 2, NPAD, 1)

    h0, hs1 = pl.pallas_call(
        _in_body,
        grid=(G,),
        in_specs=[
            pl.BlockSpec((R, F), lambda i: (i, 0)),
            pl.BlockSpec((F, H), lambda i: (0, 0)),
            pl.BlockSpec((NC, 2, R, 1), lambda i: (0, 0, i, 0)),
        ],
        out_specs=[
            pl.BlockSpec((R, H), lambda i: (i, 0)),
            pl.BlockSpec((R, H), lambda i: (i, 0)),
        ],
        out_shape=[
            jax.ShapeDtypeStruct((NPAD, H), jnp.float32),
            jax.ShapeDtypeStruct((NPAD, H), jnp.float32),
        ],
    )(xp, W_in, deg4)

    P1 = gs_call(hs1, src_r, dst_r, zeros2)

    hs2 = pl.pallas_call(
        _mid_body,
        grid=(G,),
        in_specs=[
            pl.BlockSpec((NC, R, H), lambda i: (0, i, 0)),
            pl.BlockSpec((R, H), lambda i: (i, 0)),
            pl.BlockSpec((NC, 2, R, 1), lambda i: (0, 0, i, 0)),
            pl.BlockSpec((H, H), lambda i: (0, 0)),
        ],
        out_specs=pl.BlockSpec((R, H), lambda i: (i, 0)),
        out_shape=jax.ShapeDtypeStruct((NPAD, H), jnp.float32),
    )(P1, h0, deg4, W1)

    P2 = gs_call(hs2, src_r, dst_r, zeros2)

    out = pl.pallas_call(
        _out_body,
        grid=(G,),
        in_specs=[
            pl.BlockSpec((NC, R, H), lambda i: (0, i, 0)),
            pl.BlockSpec((R, H), lambda i: (i, 0)),
            pl.BlockSpec((NC, 2, R, 1), lambda i: (0, 0, i, 0)),
            pl.BlockSpec((H, H), lambda i: (0, 0)),
            pl.BlockSpec((H, C), lambda i: (0, 0)),
        ],
        out_specs=pl.BlockSpec((R, C), lambda i: (i, 0)),
        out_shape=jax.ShapeDtypeStruct((NPAD, C), jnp.float32),
    )(P2, h0, deg4, W2, W_out)

    return out[:N]


# speedup vs baseline: 10.3171x; 10.3171x over previous
import functools

import jax
import jax.numpy as jnp
from jax import lax
from jax.experimental import pallas as pl
from jax.experimental.pallas import tpu as pltpu
from jax.experimental.pallas import tpu_sc as plsc

NC = 2
NS = 16
NW = NC * NS


def _make_degree_kernel(np2, nchunk, k):
    mesh = plsc.VectorSubcoreMesh(core_axis_name="c", subcore_axis_name="s")
    ept = 2 * np2 // NS

    @functools.partial(
        pl.kernel,
        out_type=jax.ShapeDtypeStruct((NC * 2 * np2,), jnp.float32),
        mesh=mesh,
        scratch_types=[
            pltpu.VMEM((nchunk, k), jnp.int32),
            pltpu.VMEM((k,), jnp.float32),
            pltpu.VMEM_SHARED((2 * np2,), jnp.float32),
        ],
    )
    def deg_kernel(idx_hbm, ones_hbm, zer_hbm, out_hbm, idx_v, ones_v, acc):
        cid = lax.axis_index("c")
        sid = lax.axis_index("s")
        wid = sid * NC + cid
        pltpu.sync_copy(zer_hbm.at[pl.ds(sid * ept, ept)],
                        acc.at[pl.ds(sid * ept, ept)])
        pltpu.sync_copy(idx_hbm.at[wid], idx_v)
        pltpu.sync_copy(ones_hbm, ones_v)
        plsc.subcore_barrier()

        def body(j, carry):
            pltpu.sync_copy(ones_v, acc.at[idx_v.at[j]], add=True)
            return carry

        lax.fori_loop(0, nchunk, body, 0)
        plsc.subcore_barrier()
        pltpu.sync_copy(acc.at[pl.ds(sid * ept, ept)],
                        out_hbm.at[pl.ds(cid * 2 * np2 + sid * ept, ept)])

    return deg_kernel


def _make_gather_scatter_kernel(n, hh, nchunk, k):
    mesh = plsc.VectorSubcoreMesh(core_axis_name="c", subcore_axis_name="s")
    rlo = (n // NS) // 8 * 8
    rhi = n - rlo * (NS - 1)
    nbuf = 2

    @functools.partial(
        pl.kernel,
        out_type=jax.ShapeDtypeStruct((2 * NC * n, hh), jnp.float32),
        mesh=mesh,
        scratch_types=[
            pltpu.VMEM((nchunk, k), jnp.int32),
            pltpu.VMEM((nchunk, k), jnp.int32),
            pltpu.VMEM((nbuf, k, hh), jnp.float32),
            pltpu.VMEM_SHARED((n, hh), jnp.float32),
            pltpu.SemaphoreType.DMA,
            pltpu.SemaphoreType.DMA,
        ],
        compiler_params=pltpu.CompilerParams(use_tc_tiling_on_sc=False),
    )
    def gs_kernel(hs0_hbm, hs1_hbm, src_hbm, dst_hbm, zer_hbm, out_hbm,
                  src_v, dst_v, rows_v, acc, sem0, sem1):
        cid = lax.axis_index("c")
        sid = lax.axis_index("s")
        wid = sid * NC + cid
        pltpu.sync_copy(src_hbm.at[wid], src_v)
        pltpu.sync_copy(dst_hbm.at[wid], dst_v)

        sems = (sem0, sem1)
        for half, hs_hbm in enumerate((hs0_hbm, hs1_hbm)):

            @pl.when(sid < NS - 1)
            def _zero_lo():
                pltpu.sync_copy(zer_hbm.at[pl.ds(sid * rlo, rlo)],
                                acc.at[pl.ds(sid * rlo, rlo)])

            @pl.when(sid == NS - 1)
            def _zero_hi():
                pltpu.sync_copy(zer_hbm.at[pl.ds((NS - 1) * rlo, rhi)],
                                acc.at[pl.ds((NS - 1) * rlo, rhi)])

            plsc.subcore_barrier()

            def body(j, carry):
                pltpu.async_copy(hs_hbm.at[src_v.at[j]], rows_v.at[0],
                                 sem0).wait()
                pltpu.sync_copy(rows_v.at[0], acc.at[dst_v.at[j]],
                                add=True)
                return carry

            lax.fori_loop(0, nchunk, body, 0)
            plsc.subcore_barrier()
            base = half * NC * n + cid * n

            @pl.when(sid < NS - 1)
            def _out_lo():
                pltpu.sync_copy(acc.at[pl.ds(sid * rlo, rlo)],
                                out_hbm.at[pl.ds(base + sid * rlo, rlo)])

            @pl.when(sid == NS - 1)
            def _out_hi():
                pltpu.sync_copy(
                    acc.at[pl.ds((NS - 1) * rlo, rhi)],
                    out_hbm.at[pl.ds(base + (NS - 1) * rlo, rhi)])

    return gs_kernel


def _in_body(x_ref, w_ref, deg_ref, h0_ref, hs0_ref, hs1_ref):
    ds = deg_ref[0] + deg_ref[1]
    r_out = lax.rsqrt(jnp.maximum(ds[0], 1.0))
    hm = jnp.dot(x_ref[...], w_ref[...], preferred_element_type=jnp.float32)
    h0 = jnp.maximum(hm, 0.0)
    h0_ref[...] = h0
    hs = h0 * r_out
    hh = hs.shape[1] // 2
    hs0_ref[...] = hs[:, :hh]
    hs1_ref[...] = hs[:, hh:]


def _mid_body(p0_ref, p1_ref, h0_ref, deg_ref, w_ref, hs0_ref, hs1_ref):
    ds = deg_ref[0] + deg_ref[1]
    r_out = lax.rsqrt(jnp.maximum(ds[0], 1.0))
    r_in = lax.rsqrt(jnp.maximum(ds[1], 1.0))
    agg = jnp.concatenate([p0_ref[0] + p0_ref[1],
                           p1_ref[0] + p1_ref[1]], axis=1) * r_in
    t = jnp.dot(agg + h0_ref[...], w_ref[...],
                preferred_element_type=jnp.float32)
    hs = jnp.maximum(t, 0.0) * r_out
    hh = hs.shape[1] // 2
    hs0_ref[...] = hs[:, :hh]
    hs1_ref[...] = hs[:, hh:]


def _out_body(p0_ref, p1_ref, h0_ref, deg_ref, w2_ref, wo_ref, o_ref):
    ds = deg_ref[0] + deg_ref[1]
    r_in = lax.rsqrt(jnp.maximum(ds[1], 1.0))
    agg = jnp.concatenate([p0_ref[0] + p0_ref[1],
                           p1_ref[0] + p1_ref[1]], axis=1) * r_in
    t = jnp.dot(agg + h0_ref[...], w2_ref[...],
                preferred_element_type=jnp.float32)
    o_ref[...] = jnp.dot(jnp.maximum(t, 0.0), wo_ref[...],
                         preferred_element_type=jnp.float32)


def kernel(x, edge_index, W_in, W1, W2, W_out):
    N, F = x.shape
    H = W1.shape[0]
    C = W_out.shape[1]
    E = edge_index.shape[1]
    HH = H // 2

    R = 1000
    G = N // R
    NP2 = -(-N // 1024) * 1024

    K = 80
    EW = E // NW
    NCH = EW // K
    NCH2 = 2 * E // NW // K

    src = edge_index[0]
    dst = edge_index[1]
    src_r = src.reshape(NW, NCH, K)
    dst_r = dst.reshape(NW, NCH, K)
    idx_all = jnp.concatenate([src, dst + NP2]).reshape(NW, NCH2, K)

    ones_k = jnp.ones((K,), jnp.float32)
    zeros1 = jnp.zeros((2 * NP2,), jnp.float32)
    zeros2 = jnp.zeros((N, HH), jnp.float32)

    deg_call = _make_degree_kernel(NP2, NCH2, K)
    gs_call = _make_gather_scatter_kernel(N, HH, NCH, K)

    degP = deg_call(idx_all, ones_k, zeros1)
    deg4 = degP.reshape(NC, 2, NP2, 1)

    deg_spec = pl.BlockSpec((NC, 2, R, 1), lambda i: (0, 0, i, 0))
    row_spec = pl.BlockSpec((R, H), lambda i: (i, 0))
    half_spec = pl.BlockSpec((R, HH), lambda i: (i, 0))
    p_spec = pl.BlockSpec((NC, R, HH), lambda i: (0, i, 0))

    h0, hs1a, hs1b = pl.pallas_call(
        _in_body,
        grid=(G,),
        in_specs=[
            pl.BlockSpec((R, F), lambda i: (i, 0)),
            pl.BlockSpec((F, H), lambda i: (0, 0)),
            deg_spec,
        ],
        out_specs=[row_spec, half_spec, half_spec],
        out_shape=[
            jax.ShapeDtypeStruct((N, H), jnp.float32),
            jax.ShapeDtypeStruct((N, HH), jnp.float32),
            jax.ShapeDtypeStruct((N, HH), jnp.float32),
        ],
    )(x, W_in, deg4)

    P1 = gs_call(hs1a, hs1b, src_r, dst_r, zeros2).reshape(2, NC, N, HH)

    hs2a, hs2b = pl.pallas_call(
        _mid_body,
        grid=(G,),
        in_specs=[
            p_spec,
            p_spec,
            row_spec,
            deg_spec,
            pl.BlockSpec((H, H), lambda i: (0, 0)),
        ],
        out_specs=[half_spec, half_spec],
        out_shape=[
            jax.ShapeDtypeStruct((N, HH), jnp.float32),
            jax.ShapeDtypeStruct((N, HH), jnp.float32),
        ],
    )(P1[0], P1[1], h0, deg4, W1)

    P2 = gs_call(hs2a, hs2b, src_r, dst_r, zeros2).reshape(2, NC, N, HH)

    out = pl.pallas_call(
        _out_body,
        grid=(G,),
        in_specs=[
            p_spec,
            p_spec,
            row_spec,
            deg_spec,
            pl.BlockSpec((H, H), lambda i: (0, 0)),
            pl.BlockSpec((H, C), lambda i: (0, 0)),
        ],
        out_specs=pl.BlockSpec((R, C), lambda i: (i, 0)),
        out_shape=jax.ShapeDtypeStruct((N, C), jnp.float32),
    )(P2[0], P2[1], h0, deg4, W2, W_out)

    return out


# speedup vs baseline: 18.7189x; 1.8144x over previous
import functools

import jax
import jax.numpy as jnp
from jax import lax
from jax.experimental import pallas as pl
from jax.experimental.pallas import tpu as pltpu
from jax.experimental.pallas import tpu_sc as plsc

NC = 2
NS = 16
NW = NC * NS


def _make_degree_kernel(np2, nchunk, k):
    mesh = plsc.VectorSubcoreMesh(core_axis_name="c", subcore_axis_name="s")
    ept = 2 * np2 // NS

    @functools.partial(
        pl.kernel,
        out_type=jax.ShapeDtypeStruct((NC * 2 * np2,), jnp.float32),
        mesh=mesh,
        scratch_types=[
            pltpu.VMEM((nchunk, k), jnp.int32),
            pltpu.VMEM((k,), jnp.float32),
            pltpu.VMEM_SHARED((2 * np2,), jnp.float32),
        ],
    )
    def deg_kernel(idx_hbm, ones_hbm, zer_hbm, out_hbm, idx_v, ones_v, acc):
        cid = lax.axis_index("c")
        sid = lax.axis_index("s")
        wid = sid * NC + cid
        pltpu.sync_copy(zer_hbm.at[pl.ds(sid * ept, ept)],
                        acc.at[pl.ds(sid * ept, ept)])
        pltpu.sync_copy(idx_hbm.at[wid], idx_v)
        pltpu.sync_copy(ones_hbm, ones_v)
        plsc.subcore_barrier()

        def body(j, carry):
            pltpu.sync_copy(ones_v, acc.at[idx_v.at[j]], add=True)
            return carry

        lax.fori_loop(0, nchunk, body, 0)
        plsc.subcore_barrier()
        pltpu.sync_copy(acc.at[pl.ds(sid * ept, ept)],
                        out_hbm.at[pl.ds(cid * 2 * np2 + sid * ept, ept)])

    return deg_kernel


def _make_gather_scatter_kernel(n, hh, nchunk, k):
    mesh = plsc.VectorSubcoreMesh(core_axis_name="c", subcore_axis_name="s")
    rlo = (n // NS) // 8 * 8
    rhi = n - rlo * (NS - 1)
    nbuf = 5
    assert nchunk % nbuf == 0

    @functools.partial(
        pl.kernel,
        out_type=jax.ShapeDtypeStruct((2 * NC * n, hh), jnp.float32),
        mesh=mesh,
        scratch_types=[
            pltpu.VMEM((nchunk, k), jnp.int32),
            pltpu.VMEM((nchunk, k), jnp.int32),
            pltpu.VMEM((nbuf, k, hh), jnp.float32),
            pltpu.VMEM_SHARED((n, hh), jnp.float32),
        ] + [pltpu.SemaphoreType.DMA] * nbuf,
        compiler_params=pltpu.CompilerParams(use_tc_tiling_on_sc=False),
    )
    def gs_kernel(hs0_hbm, hs1_hbm, src_hbm, dst_hbm, zer_hbm, out_hbm,
                  src_v, dst_v, rows_v, acc, *sems):
        cid = lax.axis_index("c")
        sid = lax.axis_index("s")
        wid = sid * NC + cid
        pltpu.sync_copy(src_hbm.at[wid], src_v)
        pltpu.sync_copy(dst_hbm.at[wid], dst_v)

        for half, hs_hbm in enumerate((hs0_hbm, hs1_hbm)):

            @pl.when(sid < NS - 1)
            def _zero_lo():
                pltpu.sync_copy(zer_hbm.at[pl.ds(sid * rlo, rlo)],
                                acc.at[pl.ds(sid * rlo, rlo)])

            @pl.when(sid == NS - 1)
            def _zero_hi():
                pltpu.sync_copy(zer_hbm.at[pl.ds((NS - 1) * rlo, rhi)],
                                acc.at[pl.ds((NS - 1) * rlo, rhi)])

            plsc.subcore_barrier()

            for b in range(nbuf):
                pltpu.async_copy(hs_hbm.at[src_v.at[b]], rows_v.at[b],
                                 sems[b])

            def body(j, carry):
                for b in range(nbuf):
                    cur = j * nbuf + b
                    pltpu.make_async_copy(hs_hbm.at[src_v.at[cur]],
                                          rows_v.at[b], sems[b]).wait()
                    pltpu.sync_copy(rows_v.at[b], acc.at[dst_v.at[cur]],
                                    add=True)
                    pltpu.async_copy(hs_hbm.at[src_v.at[cur + nbuf]],
                                     rows_v.at[b], sems[b])
                return carry

            lax.fori_loop(0, nchunk // nbuf - 1, body, 0)
            for b in range(nbuf):
                cur = nchunk - nbuf + b
                pltpu.make_async_copy(hs_hbm.at[src_v.at[cur]],
                                      rows_v.at[b], sems[b]).wait()
                pltpu.sync_copy(rows_v.at[b], acc.at[dst_v.at[cur]],
                                add=True)
            plsc.subcore_barrier()
            base = half * NC * n + cid * n

            @pl.when(sid < NS - 1)
            def _out_lo():
                pltpu.sync_copy(acc.at[pl.ds(sid * rlo, rlo)],
                                out_hbm.at[pl.ds(base + sid * rlo, rlo)])

            @pl.when(sid == NS - 1)
            def _out_hi():
                pltpu.sync_copy(
                    acc.at[pl.ds((NS - 1) * rlo, rhi)],
                    out_hbm.at[pl.ds(base + (NS - 1) * rlo, rhi)])

    return gs_kernel


def _in_body(x_ref, w_ref, deg_ref, h0_ref, hs0_ref, hs1_ref):
    ds = deg_ref[0] + deg_ref[1]
    r_out = lax.rsqrt(jnp.maximum(ds[0], 1.0))
    hm = jnp.dot(x_ref[...], w_ref[...], preferred_element_type=jnp.float32)
    h0 = jnp.maximum(hm, 0.0)
    h0_ref[...] = h0
    hs = h0 * r_out
    hh = hs.shape[1] // 2
    hs0_ref[...] = hs[:, :hh]
    hs1_ref[...] = hs[:, hh:]


def _mid_body(p0_ref, p1_ref, h0_ref, deg_ref, w_ref, hs0_ref, hs1_ref):
    ds = deg_ref[0] + deg_ref[1]
    r_out = lax.rsqrt(jnp.maximum(ds[0], 1.0))
    r_in = lax.rsqrt(jnp.maximum(ds[1], 1.0))
    agg = jnp.concatenate([p0_ref[0] + p0_ref[1],
                           p1_ref[0] + p1_ref[1]], axis=1) * r_in
    t = jnp.dot(agg + h0_ref[...], w_ref[...],
                preferred_element_type=jnp.float32)
    hs = jnp.maximum(t, 0.0) * r_out
    hh = hs.shape[1] // 2
    hs0_ref[...] = hs[:, :hh]
    hs1_ref[...] = hs[:, hh:]


def _out_body(p0_ref, p1_ref, h0_ref, deg_ref, w2_ref, wo_ref, o_ref):
    ds = deg_ref[0] + deg_ref[1]
    r_in = lax.rsqrt(jnp.maximum(ds[1], 1.0))
    agg = jnp.concatenate([p0_ref[0] + p0_ref[1],
                           p1_ref[0] + p1_ref[1]], axis=1) * r_in
    t = jnp.dot(agg + h0_ref[...], w2_ref[...],
                preferred_element_type=jnp.float32)
    o_ref[...] = jnp.dot(jnp.maximum(t, 0.0), wo_ref[...],
                         preferred_element_type=jnp.float32)


def kernel(x, edge_index, W_in, W1, W2, W_out):
    N, F = x.shape
    H = W1.shape[0]
    C = W_out.shape[1]
    E = edge_index.shape[1]
    HH = H // 2

    R = 1000
    G = N // R
    NP2 = -(-N // 1024) * 1024

    K = 80
    EW = E // NW
    NCH = EW // K
    NCH2 = 2 * E // NW // K

    src = edge_index[0]
    dst = edge_index[1]
    src_r = src.reshape(NW, NCH, K)
    dst_r = dst.reshape(NW, NCH, K)
    idx_all = jnp.concatenate([src, dst + NP2]).reshape(NW, NCH2, K)

    ones_k = jnp.ones((K,), jnp.float32)
    zeros1 = jnp.zeros((2 * NP2,), jnp.float32)
    zeros2 = jnp.zeros((N, HH), jnp.float32)

    deg_call = _make_degree_kernel(NP2, NCH2, K)
    gs_call = _make_gather_scatter_kernel(N, HH, NCH, K)

    degP = deg_call(idx_all, ones_k, zeros1)
    deg4 = degP.reshape(NC, 2, NP2, 1)

    deg_spec = pl.BlockSpec((NC, 2, R, 1), lambda i: (0, 0, i, 0))
    row_spec = pl.BlockSpec((R, H), lambda i: (i, 0))
    half_spec = pl.BlockSpec((R, HH), lambda i: (i, 0))
    p_spec = pl.BlockSpec((NC, R, HH), lambda i: (0, i, 0))

    h0, hs1a, hs1b = pl.pallas_call(
        _in_body,
        grid=(G,),
        in_specs=[
            pl.BlockSpec((R, F), lambda i: (i, 0)),
            pl.BlockSpec((F, H), lambda i: (0, 0)),
            deg_spec,
        ],
        out_specs=[row_spec, half_spec, half_spec],
        out_shape=[
            jax.ShapeDtypeStruct((N, H), jnp.float32),
            jax.ShapeDtypeStruct((N, HH), jnp.float32),
            jax.ShapeDtypeStruct((N, HH), jnp.float32),
        ],
    )(x, W_in, deg4)

    P1 = gs_call(hs1a, hs1b, src_r, dst_r, zeros2).reshape(2, NC, N, HH)

    hs2a, hs2b = pl.pallas_call(
        _mid_body,
        grid=(G,),
        in_specs=[
            p_spec,
            p_spec,
            row_spec,
            deg_spec,
            pl.BlockSpec((H, H), lambda i: (0, 0)),
        ],
        out_specs=[half_spec, half_spec],
        out_shape=[
            jax.ShapeDtypeStruct((N, HH), jnp.float32),
            jax.ShapeDtypeStruct((N, HH), jnp.float32),
        ],
    )(P1[0], P1[1], h0, deg4, W1)

    P2 = gs_call(hs2a, hs2b, src_r, dst_r, zeros2).reshape(2, NC, N, HH)

    out = pl.pallas_call(
        _out_body,
        grid=(G,),
        in_specs=[
            p_spec,
            p_spec,
            row_spec,
            deg_spec,
            pl.BlockSpec((H, H), lambda i: (0, 0)),
            pl.BlockSpec((H, C), lambda i: (0, 0)),
        ],
        out_specs=pl.BlockSpec((R, C), lambda i: (i, 0)),
        out_shape=jax.ShapeDtypeStruct((N, C), jnp.float32),
    )(P2[0], P2[1], h0, deg4, W2, W_out)

    return out


# speedup vs baseline: 19.5644x; 1.0452x over previous
import functools

import jax
import jax.numpy as jnp
from jax import lax
from jax.experimental import pallas as pl
from jax.experimental.pallas import tpu as pltpu
from jax.experimental.pallas import tpu_sc as plsc

NC = 2
NS = 16
NW = NC * NS


def _make_degree_kernel(np2, nchunk, k):
    mesh = plsc.VectorSubcoreMesh(core_axis_name="c", subcore_axis_name="s")
    ept = 2 * np2 // NS
    kv = k // 16

    @functools.partial(
        pl.kernel,
        out_type=jax.ShapeDtypeStruct((NC * 2 * np2,), jnp.float32),
        mesh=mesh,
        scratch_types=[
            pltpu.VMEM((nchunk, k), jnp.int32),
            pltpu.VMEM((nchunk, k), jnp.int32),
            pltpu.VMEM((k,), jnp.float32),
            pltpu.VMEM_SHARED((2 * np2,), jnp.float32),
        ],
    )
    def deg_kernel(src_hbm, dst_hbm, ones_hbm, zer_hbm, out_hbm,
                   sv, dv, ones_v, acc):
        cid = lax.axis_index("c")
        sid = lax.axis_index("s")
        wid = sid * NC + cid
        pltpu.sync_copy(zer_hbm.at[pl.ds(sid * ept, ept)],
                        acc.at[pl.ds(sid * ept, ept)])
        pltpu.sync_copy(src_hbm.at[wid], sv)
        pltpu.sync_copy(dst_hbm.at[wid], dv)
        pltpu.sync_copy(ones_hbm, ones_v)

        def addoff(j, carry):
            for c in range(kv):
                sl = pl.ds(c * 16, 16)
                dv[j, sl] = dv[j, sl] + np2
            return carry

        lax.fori_loop(0, nchunk, addoff, 0)
        plsc.subcore_barrier()

        def b_src(j, carry):
            pltpu.sync_copy(ones_v, acc.at[sv.at[j]], add=True)
            return carry

        def b_dst(j, carry):
            pltpu.sync_copy(ones_v, acc.at[dv.at[j]], add=True)
            return carry

        lax.fori_loop(0, nchunk, b_src, 0)
        lax.fori_loop(0, nchunk, b_dst, 0)
        plsc.subcore_barrier()
        pltpu.sync_copy(acc.at[pl.ds(sid * ept, ept)],
                        out_hbm.at[pl.ds(cid * 2 * np2 + sid * ept, ept)])

    return deg_kernel


def _make_gather_scatter_kernel(n, npad, hh, nchunk, k):
    mesh = plsc.VectorSubcoreMesh(core_axis_name="c", subcore_axis_name="s")
    rlo = (n // NS) // 8 * 8
    rhi = n - rlo * (NS - 1)
    nbuf = 5
    kv = k // 16
    assert nchunk % nbuf == 0

    @functools.partial(
        pl.kernel,
        out_type=jax.ShapeDtypeStruct((2 * NC * npad, hh), jnp.float32),
        mesh=mesh,
        scratch_types=[
            pltpu.VMEM((nchunk, k), jnp.int32),
            pltpu.VMEM((nchunk, k), jnp.int32),
            pltpu.VMEM((nchunk, k), jnp.int32),
            pltpu.VMEM((nbuf, k, hh), jnp.float32),
            pltpu.VMEM_SHARED((n, hh), jnp.float32),
        ] + [pltpu.SemaphoreType.DMA] * nbuf,
        compiler_params=pltpu.CompilerParams(use_tc_tiling_on_sc=False),
    )
    def gs_kernel(hs_hbm, src_hbm, dst_hbm, zer_hbm, out_hbm,
                  src_v, src2_v, dst_v, rows_v, acc, *gsem):
        cid = lax.axis_index("c")
        sid = lax.axis_index("s")
        wid = sid * NC + cid
        pltpu.sync_copy(src_hbm.at[wid], src_v)
        pltpu.sync_copy(dst_hbm.at[wid], dst_v)

        def dbl(j, carry):
            for c in range(kv):
                sl = pl.ds(c * 16, 16)
                src2_v[j, sl] = src_v[j, sl] * 2
            return carry

        def inc(j, carry):
            for c in range(kv):
                sl = pl.ds(c * 16, 16)
                src2_v[j, sl] = src2_v[j, sl] + 1
            return carry

        lax.fori_loop(0, nchunk, dbl, 0)

        for half in range(2):
            if half == 1:
                lax.fori_loop(0, nchunk, inc, 0)

            @pl.when(sid < NS - 1)
            def _zero_lo():
                pltpu.sync_copy(zer_hbm.at[pl.ds(sid * rlo, rlo)],
                                acc.at[pl.ds(sid * rlo, rlo)])

            @pl.when(sid == NS - 1)
            def _zero_hi():
                pltpu.sync_copy(zer_hbm.at[pl.ds((NS - 1) * rlo, rhi)],
                                acc.at[pl.ds((NS - 1) * rlo, rhi)])

            plsc.subcore_barrier()

            for b in range(nbuf):
                pltpu.async_copy(hs_hbm.at[src2_v.at[b]], rows_v.at[b],
                                 gsem[b])

            def body(j, carry):
                for b in range(nbuf):
                    cur = j * nbuf + b
                    pltpu.make_async_copy(hs_hbm.at[src2_v.at[cur]],
                                          rows_v.at[b], gsem[b]).wait()
                    pltpu.sync_copy(rows_v.at[b], acc.at[dst_v.at[cur]],
                                    add=True)
                    pltpu.async_copy(hs_hbm.at[src2_v.at[cur + nbuf]],
                                     rows_v.at[b], gsem[b])
                return carry

            lax.fori_loop(0, nchunk // nbuf - 1, body, 0)
            for b in range(nbuf):
                cur = nchunk - nbuf + b
                pltpu.make_async_copy(hs_hbm.at[src2_v.at[cur]],
                                      rows_v.at[b], gsem[b]).wait()
                pltpu.sync_copy(rows_v.at[b], acc.at[dst_v.at[cur]],
                                add=True)
            plsc.subcore_barrier()
            base = (half * NC + cid) * npad

            @pl.when(sid < NS - 1)
            def _out_lo():
                pltpu.sync_copy(acc.at[pl.ds(sid * rlo, rlo)],
                                out_hbm.at[pl.ds(base + sid * rlo, rlo)])

            @pl.when(sid == NS - 1)
            def _out_hi():
                pltpu.sync_copy(
                    acc.at[pl.ds((NS - 1) * rlo, rhi)],
                    out_hbm.at[pl.ds(base + (NS - 1) * rlo, rhi)])

    return gs_kernel


def _rsqrt_cols(deg, r):
    dout = deg[0, 0] + deg[1, 0]
    din = deg[0, 1] + deg[1, 1]
    r_out = lax.rsqrt(jnp.maximum(dout, 1.0))
    r_in = lax.rsqrt(jnp.maximum(din, 1.0))
    return r_out, r_in


def _in_body(x_ref, w_ref, deg_ref, h0_ref, hs_ref):
    r_out, _ = _rsqrt_cols(deg_ref[...], x_ref.shape[0])
    hm = jnp.dot(x_ref[...], w_ref[...], preferred_element_type=jnp.float32)
    h0 = jnp.maximum(hm, 0.0)
    h0_ref[...] = h0
    hs_ref[...] = h0 * r_out


def _mid_body(p0_ref, p1_ref, h0_ref, deg_ref, w_ref, hs_ref):
    r_out, r_in = _rsqrt_cols(deg_ref[...], h0_ref.shape[0])
    agg = jnp.concatenate([p0_ref[0] + p0_ref[1],
                           p1_ref[0] + p1_ref[1]], axis=1) * r_in
    t = jnp.dot(agg + h0_ref[...], w_ref[...],
                preferred_element_type=jnp.float32)
    hs_ref[...] = jnp.maximum(t, 0.0) * r_out


def _out_body(p0_ref, p1_ref, h0_ref, deg_ref, w2_ref, wo_ref, o_ref):
    _, r_in = _rsqrt_cols(deg_ref[...], h0_ref.shape[0])
    agg = jnp.concatenate([p0_ref[0] + p0_ref[1],
                           p1_ref[0] + p1_ref[1]], axis=1) * r_in
    t = jnp.dot(agg + h0_ref[...], w2_ref[...],
                preferred_element_type=jnp.float32)
    o_ref[...] = jnp.dot(jnp.maximum(t, 0.0), wo_ref[...],
                         preferred_element_type=jnp.float32)


def kernel(x, edge_index, W_in, W1, W2, W_out):
    N, F = x.shape
    H = W1.shape[0]
    C = W_out.shape[1]
    E = edge_index.shape[1]
    HH = H // 2

    R = 1000
    G = N // R
    NP2 = -(-N // 1024) * 1024

    K = 80
    NCH = E // NW // K

    src = edge_index[0]
    dst = edge_index[1]
    src_r = src.reshape(NW, NCH, K)
    dst_r = dst.reshape(NW, NCH, K)

    ones_k = jnp.ones((K,), jnp.float32)
    zeros1 = jnp.zeros((2 * NP2,), jnp.float32)
    zeros2 = jnp.zeros((N, HH), jnp.float32)

    deg_call = _make_degree_kernel(NP2, NCH, K)
    gs_call = _make_gather_scatter_kernel(N, N, HH, NCH, K)

    degP = deg_call(src_r, dst_r, ones_k, zeros1)
    deg4 = degP.reshape(NC, 2, NP2, 1)

    deg_spec = pl.BlockSpec((NC, 2, R, 1), lambda i: (0, 0, i, 0))
    row_spec = pl.BlockSpec((R, H), lambda i: (i, 0))
    p_spec = pl.BlockSpec((NC, R, HH), lambda i: (0, i, 0))

    h0, hs1 = pl.pallas_call(
        _in_body,
        grid=(G,),
        in_specs=[
            pl.BlockSpec((R, F), lambda i: (i, 0)),
            pl.BlockSpec((F, H), lambda i: (0, 0)),
            deg_spec,
        ],
        out_specs=[row_spec, row_spec],
        out_shape=[
            jax.ShapeDtypeStruct((N, H), jnp.float32),
            jax.ShapeDtypeStruct((N, H), jnp.float32),
        ],
    )(x, W_in, deg4)

    P1 = gs_call(hs1.reshape(2 * N, HH), src_r, dst_r,
                 zeros2).reshape(2, NC, N, HH)

    hs2 = pl.pallas_call(
        _mid_body,
        grid=(G,),
        in_specs=[
            p_spec,
            p_spec,
            row_spec,
            deg_spec,
            pl.BlockSpec((H, H), lambda i: (0, 0)),
        ],
        out_specs=row_spec,
        out_shape=jax.ShapeDtypeStruct((N, H), jnp.float32),
    )(P1[0], P1[1], h0, deg4, W1)

    P2 = gs_call(hs2.reshape(2 * N, HH), src_r, dst_r,
                 zeros2).reshape(2, NC, N, HH)

    out = pl.pallas_call(
        _out_body,
        grid=(G,),
        in_specs=[
            p_spec,
            p_spec,
            row_spec,
            deg_spec,
            pl.BlockSpec((H, H), lambda i: (0, 0)),
            pl.BlockSpec((H, C), lambda i: (0, 0)),
        ],
        out_specs=pl.BlockSpec((R, C), lambda i: (i, 0)),
        out_shape=jax.ShapeDtypeStruct((N, C), jnp.float32),
    )(P2[0], P2[1], h0, deg4, W2, W_out)

    return out
